# baseline (speedup 1.0000x reference)
import jax
import jax.numpy as jnp
from jax import lax
from jax.experimental import pallas as pl
from jax.experimental.pallas import tpu as pltpu
from jax.experimental.pallas import tpu_sc as plsc

B = 16384
D = 64
MARGIN = 2.0
L = 16
NC, NS = 2, 16
NW = NC * NS
BPW = B // NW
C = 128
NCHUNK = BPW // C


def _body(heads, relations, tails, h_hat, t_hat, ent, rel, out_hbm,
          idx_h, idx_r, idx_t, idx_hh, idx_th,
          rows_h, rows_r, rows_t, rows_hh, rows_th, out_v, sem):
    wid = lax.axis_index("s") * NC + lax.axis_index("c")
    base = wid * BPW

    def chunk(ci, carry):
        off = base + ci * C
        pltpu.sync_copy(heads.at[pl.ds(off, C)], idx_h)
        pltpu.sync_copy(relations.at[pl.ds(off, C)], idx_r)
        pltpu.sync_copy(tails.at[pl.ds(off, C)], idx_t)
        pltpu.sync_copy(h_hat.at[pl.ds(off, C)], idx_hh)
        pltpu.sync_copy(t_hat.at[pl.ds(off, C)], idx_th)
        cps = [
            pltpu.async_copy(ent.at[idx_h], rows_h, sem),
            pltpu.async_copy(rel.at[idx_r], rows_r, sem),
            pltpu.async_copy(ent.at[idx_t], rows_t, sem),
            pltpu.async_copy(ent.at[idx_hh], rows_hh, sem),
            pltpu.async_copy(ent.at[idx_th], rows_th, sem),
        ]
        for cp in cps:
            cp.wait()

        def group(g, gcarry):
            row_ids = g * L + lax.iota(jnp.int32, L)
            d1 = jnp.zeros((L,), jnp.float32)
            d2 = jnp.zeros((L,), jnp.float32)
            for j in range(D):
                col = jnp.full((L,), j, jnp.int32)
                rv = plsc.load_gather(rows_r, [row_ids, col])
                hv = plsc.load_gather(rows_h, [row_ids, col])
                tv = plsc.load_gather(rows_t, [row_ids, col])
                hhv = plsc.load_gather(rows_hh, [row_ids, col])
                thv = plsc.load_gather(rows_th, [row_ids, col])
                d1 = d1 + jnp.abs(hv + rv - tv)
                d2 = d2 + jnp.abs(hhv + rv - thv)
            m = jnp.maximum(MARGIN + d1 - d2, 0.0)
            out_v[pl.ds(ci * C + g * L, L)] = m
            return gcarry

        lax.fori_loop(0, C // L, group, 0)
        return carry

    lax.fori_loop(0, NCHUNK, chunk, 0)
    pltpu.sync_copy(out_v, out_hbm.at[pl.ds(base, BPW)])


@jax.jit
def kernel(heads, relations, tails, h_hat, t_hat, entity_weight, rel_weight):
    mesh = plsc.VectorSubcoreMesh(core_axis_name="c", subcore_axis_name="s")
    fn = pl.kernel(
        _body,
        out_type=jax.ShapeDtypeStruct((B,), jnp.float32),
        mesh=mesh,
        compiler_params=pltpu.CompilerParams(
            needs_layout_passes=False, use_tc_tiling_on_sc=False
        ),
        scratch_types=[
            pltpu.VMEM((C,), jnp.int32),
            pltpu.VMEM((C,), jnp.int32),
            pltpu.VMEM((C,), jnp.int32),
            pltpu.VMEM((C,), jnp.int32),
            pltpu.VMEM((C,), jnp.int32),
            pltpu.VMEM((C, D), jnp.float32),
            pltpu.VMEM((C, D), jnp.float32),
            pltpu.VMEM((C, D), jnp.float32),
            pltpu.VMEM((C, D), jnp.float32),
            pltpu.VMEM((C, D), jnp.float32),
            pltpu.VMEM((BPW,), jnp.float32),
            pltpu.SemaphoreType.DMA,
        ],
    )
    out = fn(heads, relations, tails, h_hat, t_hat, entity_weight, rel_weight)
    return out[:, None]


# speedup vs baseline: 1.0490x; 1.0490x over previous
import functools

import jax
import jax.numpy as jnp
from jax import lax
from jax.experimental import pallas as pl
from jax.experimental.pallas import tpu as pltpu
from jax.experimental.pallas import tpu_sc as plsc

B = 16384
D = 64
MARGIN = 2.0
L = 16
NC, NS = 2, 16
NW = NC * NS
BPW = B // NW
C = 128
NCHUNK = BPW // C


def _body(heads, relations, tails, h_hat, t_hat, ent, rel, out_hbm,
          idx_h, idx_r, idx_t, idx_hh, idx_th,
          rows_h, rows_r, rows_t, rows_hh, rows_th, out_v, sem):
    wid = lax.axis_index("s") * NC + lax.axis_index("c")
    base = wid * BPW

    def chunk(ci, carry):
        off = base + ci * C
        icps = [
            pltpu.async_copy(heads.at[pl.ds(off, C)], idx_h, sem),
            pltpu.async_copy(relations.at[pl.ds(off, C)], idx_r, sem),
            pltpu.async_copy(tails.at[pl.ds(off, C)], idx_t, sem),
            pltpu.async_copy(h_hat.at[pl.ds(off, C)], idx_hh, sem),
            pltpu.async_copy(t_hat.at[pl.ds(off, C)], idx_th, sem),
        ]
        for cp in icps:
            cp.wait()
        cps = [
            pltpu.async_copy(ent.at[idx_h], rows_h, sem),
            pltpu.async_copy(rel.at[idx_r], rows_r, sem),
            pltpu.async_copy(ent.at[idx_t], rows_t, sem),
            pltpu.async_copy(ent.at[idx_hh], rows_hh, sem),
            pltpu.async_copy(ent.at[idx_th], rows_th, sem),
        ]
        for cp in cps:
            cp.wait()

        def group(g, gcarry):
            row_ids = g * L + lax.iota(jnp.int32, L)
            zero = jnp.zeros((L,), jnp.float32)

            def cols(j, dcarry):
                d1, d2 = dcarry
                col = jnp.full((L,), j, jnp.int32)
                rv = plsc.load_gather(rows_r, [row_ids, col])
                hv = plsc.load_gather(rows_h, [row_ids, col])
                tv = plsc.load_gather(rows_t, [row_ids, col])
                hhv = plsc.load_gather(rows_hh, [row_ids, col])
                thv = plsc.load_gather(rows_th, [row_ids, col])
                d1 = d1 + jnp.abs(hv + rv - tv)
                d2 = d2 + jnp.abs(hhv + rv - thv)
                return (d1, d2)

            d1, d2 = plsc.parallel_loop(0, D, 1, unroll=4, carry=(zero, zero))(cols)
            m = jnp.maximum(MARGIN + d1 - d2, 0.0)
            out_v[pl.ds(ci * C + g * L, L)] = m
            return gcarry

        lax.fori_loop(0, C // L, group, 0)
        return carry

    lax.fori_loop(0, NCHUNK, chunk, 0)
    pltpu.sync_copy(out_v, out_hbm.at[pl.ds(base, BPW)])


@jax.jit
def kernel(heads, relations, tails, h_hat, t_hat, entity_weight, rel_weight):
    mesh = plsc.VectorSubcoreMesh(core_axis_name="c", subcore_axis_name="s")
    fn = pl.kernel(
        _body,
        out_type=jax.ShapeDtypeStruct((B,), jnp.float32),
        mesh=mesh,
        compiler_params=pltpu.CompilerParams(
            needs_layout_passes=False, use_tc_tiling_on_sc=False
        ),
        scratch_types=[
            pltpu.VMEM((C,), jnp.int32),
            pltpu.VMEM((C,), jnp.int32),
            pltpu.VMEM((C,), jnp.int32),
            pltpu.VMEM((C,), jnp.int32),
            pltpu.VMEM((C,), jnp.int32),
            pltpu.VMEM((C, D), jnp.float32),
            pltpu.VMEM((C, D), jnp.float32),
            pltpu.VMEM((C, D), jnp.float32),
            pltpu.VMEM((C, D), jnp.float32),
            pltpu.VMEM((C, D), jnp.float32),
            pltpu.VMEM((BPW,), jnp.float32),
            pltpu.SemaphoreType.DMA,
        ],
    )
    out = fn(heads, relations, tails, h_hat, t_hat, entity_weight, rel_weight)
    return out[:, None]
